# baseline (speedup 1.0000x reference)
import functools

import jax
import jax.numpy as jnp
from jax import lax
from jax.experimental import pallas as pl
from jax.experimental.pallas import tpu as pltpu
from jax.experimental.pallas import tpu_sc as plsc

EPS = 1e-5

NC = 2
NS = 16
NW = NC * NS

K = 200
ZR = 40
WT = 10


def _build_idx(dst_ref, out_ref, base, count, scale, offset):
    nfull = count // 16
    for i in range(nfull):
        v = dst_ref[pl.ds(base + i * 16, 16)]
        out_ref[pl.ds(i * 16, 16)] = v * scale + offset
    if count % 16 != 0:
        o = count - 16
        v = dst_ref[pl.ds(base + o, 16)]
        out_ref[pl.ds(o, 16)] = v * scale + offset


def _make_spmm(N, E, S, interpret=False):
    EPT = E // NW
    KC = 96
    FULL = EPT // KC
    TAILB = FULL * KC
    PAIRS = (FULL - 2) // 2
    RPT = N // WT
    mesh = plsc.VectorSubcoreMesh(core_axis_name="c", subcore_axis_name="s")

    @functools.partial(
        pl.kernel,
        out_type=jax.ShapeDtypeStruct((NC * S * N, 128), jnp.float32),
        mesh=mesh,
        interpret=interpret,
        scratch_types=[
            pltpu.VMEM((EPT + 16,), jnp.int32),
            pltpu.VMEM((EPT + 16,), jnp.int32),
            pltpu.VMEM((KC,), jnp.int32),
            pltpu.VMEM((KC,), jnp.int32),
            pltpu.VMEM((KC,), jnp.int32),
            pltpu.VMEM((KC,), jnp.int32),
            pltpu.VMEM((KC, 128), jnp.float32),
            pltpu.VMEM((KC, 128), jnp.float32),
            pltpu.VMEM((ZR, 128), jnp.float32),
            pltpu.VMEM_SHARED((N + 8, 128), jnp.float32),
            pltpu.SemaphoreType.DMA,
            pltpu.SemaphoreType.DMA,
        ],
    )
    def spmm(table, src, dst, zeros, out,
             src_all, dst_all, gidxA, gidxB, sidxA, sidxB,
             rowsA, rowsB, zbuf, acc, semA, semB):
        cid = lax.axis_index("c")
        sid = lax.axis_index("s")
        wid = cid * NS + sid
        ebase = wid * EPT
        pltpu.sync_copy(src.at[pl.ds(ebase, EPT)], src_all.at[pl.ds(0, EPT)])
        pltpu.sync_copy(dst.at[pl.ds(ebase, EPT)], dst_all.at[pl.ds(0, EPT)])
        pltpu.sync_copy(zeros, zbuf)

        for s in range(S):

            def build(g, gidx, sidx):
                for i in range(KC // 16):
                    o = g * KC + i * 16
                    gidx[pl.ds(i * 16, 16)] = src_all[pl.ds(o, 16)] * S + s
                    sidx[pl.ds(i * 16, 16)] = dst_all[pl.ds(o, 16)]

            def build_tail(gidx, sidx):
                lane = lax.iota(jnp.int32, 16)
                m = lane < (EPT - TAILB)
                sv = src_all[pl.ds(TAILB, 16)]
                dv = dst_all[pl.ds(TAILB, 16)]
                gidx[pl.ds(0, 16)] = jnp.where(m, sv * S + s, 0)
                sidx[pl.ds(0, 16)] = jnp.where(m, dv, N)
                for i in range(1, KC // 16):
                    gidx[pl.ds(i * 16, 16)] = jnp.zeros((16,), jnp.int32)
                    sidx[pl.ds(i * 16, 16)] = jnp.full((16,), N, jnp.int32)

            def start(gidx, rows, sem):
                return pltpu.async_copy(table.at[gidx], rows, sem)

            def wait(gidx, rows, sem):
                pltpu.make_async_copy(table.at[gidx], rows, sem).wait()

            def scatter(rows, sidx):
                pltpu.sync_copy(rows, acc.at[sidx], add=True)

            plsc.subcore_barrier()

            @pl.when(sid < WT)
            def _zero():
                for z in range(RPT // ZR):
                    pltpu.sync_copy(zbuf,
                                    acc.at[pl.ds(sid * RPT + z * ZR, ZR)])

            plsc.subcore_barrier()

            build(0, gidxA, sidxA)
            start(gidxA, rowsA, semA)

            def pair(i, _):
                g = 2 * i
                build(g + 1, gidxB, sidxB)
                start(gidxB, rowsB, semB)
                wait(gidxA, rowsA, semA)
                scatter(rowsA, sidxA)
                build(g + 2, gidxA, sidxA)
                start(gidxA, rowsA, semA)
                wait(gidxB, rowsB, semB)
                scatter(rowsB, sidxB)
                return 0

            lax.fori_loop(0, PAIRS, pair, 0)
            build(FULL - 1, gidxB, sidxB)
            start(gidxB, rowsB, semB)
            wait(gidxA, rowsA, semA)
            scatter(rowsA, sidxA)
            build_tail(gidxA, sidxA)
            start(gidxA, rowsA, semA)
            wait(gidxB, rowsB, semB)
            scatter(rowsB, sidxB)
            wait(gidxA, rowsA, semA)
            scatter(rowsA, sidxA)

            plsc.subcore_barrier()

            @pl.when(sid < WT)
            def _writeout():
                obase = (cid * S + s) * N + sid * RPT
                for z in range(RPT // ZR):
                    pltpu.sync_copy(acc.at[pl.ds(sid * RPT + z * ZR, ZR)],
                                    out.at[pl.ds(obase + z * ZR, ZR)])

    return spmm


def _make_deg(N, E, interpret=False):
    EPT = E // NW
    CHUNKS = EPT // K
    RPT = N // WT
    mesh = plsc.VectorSubcoreMesh(core_axis_name="c", subcore_axis_name="s")

    @functools.partial(
        pl.kernel,
        out_type=jax.ShapeDtypeStruct((NC * N, 128), jnp.float32),
        mesh=mesh,
        interpret=interpret,
        scratch_types=[
            pltpu.VMEM((EPT,), jnp.int32),
            pltpu.VMEM((K,), jnp.int32),
            pltpu.VMEM((K, 128), jnp.float32),
            pltpu.VMEM((ZR, 128), jnp.float32),
            pltpu.VMEM_SHARED((N, 128), jnp.float32),
        ],
    )
    def deg(dst, ones, zeros, out, dst_all, sidx, obuf, zbuf, acc):
        cid = lax.axis_index("c")
        sid = lax.axis_index("s")
        wid = cid * NS + sid
        pltpu.sync_copy(dst.at[pl.ds(wid * EPT, EPT)], dst_all)
        pltpu.sync_copy(ones, obuf)
        pltpu.sync_copy(zeros, zbuf)
        plsc.subcore_barrier()

        @pl.when(sid < WT)
        def _zero():
            for z in range(RPT // ZR):
                pltpu.sync_copy(zbuf, acc.at[pl.ds(sid * RPT + z * ZR, ZR)])

        plsc.subcore_barrier()

        def chunk(g, _):
            _build_idx(dst_all, sidx, g * K, K, 1, 0)
            pltpu.sync_copy(obuf, acc.at[sidx], add=True)
            return 0

        lax.fori_loop(0, CHUNKS, chunk, 0)
        plsc.subcore_barrier()

        @pl.when(sid < WT)
        def _writeout():
            obase = cid * N + sid * RPT
            for z in range(RPT // ZR):
                pltpu.sync_copy(acc.at[pl.ds(sid * RPT + z * ZR, ZR)],
                                out.at[pl.ds(obase + z * ZR, ZR)])

    return deg


def _inv_deg(dp):
    deg = dp[0, :, 0:1] + dp[1, :, 0:1]
    return jnp.where(deg > 0.0, 1.0 / jnp.maximum(deg, 1.0), 0.0)


def _tc_layer(parts, degp, xin, wa, wx, b, g, bn, *, ln_relu, R=400,
              interpret=False):
    N, C = xin.shape
    S = C // 128
    H = wa.shape[1]

    def body(p_ref, d_ref, x_ref, wa_ref, wx_ref, b_ref, g_ref, bn_ref, o_ref):
        p = p_ref[...]
        ps = p[0] + p[1]
        inv = _inv_deg(d_ref[...])
        agg = jnp.concatenate([ps[s] for s in range(S)], axis=-1) * inv
        h = (jnp.dot(agg, wa_ref[...], preferred_element_type=jnp.float32)
             + jnp.dot(x_ref[...], wx_ref[...],
                       preferred_element_type=jnp.float32)
             + b_ref[...])
        if ln_relu:
            mu = jnp.mean(h, axis=-1, keepdims=True)
            var = jnp.mean((h - mu) ** 2, axis=-1, keepdims=True)
            h = (h - mu) * lax.rsqrt(var + EPS) * g_ref[...] + bn_ref[...]
            h = jnp.maximum(h, 0.0)
        o_ref[...] = h

    return pl.pallas_call(
        body,
        grid=(N // R,),
        in_specs=[
            pl.BlockSpec((NC, S, R, 128), lambda i: (0, 0, i, 0)),
            pl.BlockSpec((NC, R, 128), lambda i: (0, i, 0)),
            pl.BlockSpec((R, C), lambda i: (i, 0)),
            pl.BlockSpec((C, H), lambda i: (0, 0)),
            pl.BlockSpec((C, H), lambda i: (0, 0)),
            pl.BlockSpec((1, H), lambda i: (0, 0)),
            pl.BlockSpec((1, H), lambda i: (0, 0)),
            pl.BlockSpec((1, H), lambda i: (0, 0)),
        ],
        out_specs=pl.BlockSpec((R, H), lambda i: (i, 0)),
        out_shape=jax.ShapeDtypeStruct((N, H), jnp.float32),
        interpret=interpret,
    )(parts, degp, xin, wa, wx, b, g, bn)


def _tc_lin2(xin, wa, wx, b, *, R=400, interpret=False):
    N, C = xin.shape
    H = wa.shape[1]

    def body(x_ref, wa_ref, wx_ref, b_ref, ya_ref, yx_ref):
        xb = x_ref[...]
        ya_ref[...] = jnp.dot(xb, wa_ref[...],
                              preferred_element_type=jnp.float32)
        yx_ref[...] = jnp.dot(xb, wx_ref[...],
                              preferred_element_type=jnp.float32) + b_ref[...]

    return pl.pallas_call(
        body,
        grid=(N // R,),
        in_specs=[
            pl.BlockSpec((R, C), lambda i: (i, 0)),
            pl.BlockSpec((C, H), lambda i: (0, 0)),
            pl.BlockSpec((C, H), lambda i: (0, 0)),
            pl.BlockSpec((1, H), lambda i: (0, 0)),
        ],
        out_specs=[
            pl.BlockSpec((R, H), lambda i: (i, 0)),
            pl.BlockSpec((R, H), lambda i: (i, 0)),
        ],
        out_shape=[
            jax.ShapeDtypeStruct((N, H), jnp.float32),
            jax.ShapeDtypeStruct((N, H), jnp.float32),
        ],
        interpret=interpret,
    )(xin, wa, wx, b)


def _tc_final(parts, degp, yx, *, R=400, interpret=False):
    N, H = yx.shape
    S = H // 128

    def body(p_ref, d_ref, y_ref, o_ref):
        p = p_ref[...]
        ps = p[0] + p[1]
        inv = _inv_deg(d_ref[...])
        agg = jnp.concatenate([ps[s] for s in range(S)], axis=-1) * inv
        o_ref[...] = agg + y_ref[...]

    return pl.pallas_call(
        body,
        grid=(N // R,),
        in_specs=[
            pl.BlockSpec((NC, S, R, 128), lambda i: (0, 0, i, 0)),
            pl.BlockSpec((NC, R, 128), lambda i: (0, i, 0)),
            pl.BlockSpec((R, H), lambda i: (i, 0)),
        ],
        out_specs=pl.BlockSpec((R, H), lambda i: (i, 0)),
        out_shape=jax.ShapeDtypeStruct((N, H), jnp.float32),
        interpret=interpret,
    )(parts, degp, yx)


def kernel(x, edge_index, W0, b0, W1, b1, W2, b2, g0, bn0, g1, bn1):
    N, C0 = x.shape
    E = edge_index.shape[1]
    H = W0.shape[0]
    OUT = W2.shape[0]
    src = edge_index[0].astype(jnp.int32)
    dst = edge_index[1].astype(jnp.int32)

    Wt0 = W0.T
    Wt1 = W1.T
    Wt2 = W2.T
    wa0, wx0 = Wt0[:C0], Wt0[C0:]
    wa1, wx1 = Wt1[:H], Wt1[H:]
    wa2, wx2 = Wt2[:H], Wt2[H:]
    b0r, g0r, bn0r = b0.reshape(1, -1), g0.reshape(1, -1), bn0.reshape(1, -1)
    b1r, g1r, bn1r = b1.reshape(1, -1), g1.reshape(1, -1), bn1.reshape(1, -1)
    b2r = b2.reshape(1, -1)

    z128 = jnp.zeros((ZR, 128), jnp.float32)
    o128 = jnp.ones((K, 128), jnp.float32)

    S0 = C0 // 128
    SH = H // 128
    SO = OUT // 128

    degp = _make_deg(N, E)(dst, o128, z128).reshape(NC, N, 128)

    p0 = _make_spmm(N, E, S0)(x.reshape(-1, 128), src, dst, z128)
    h0 = _tc_layer(p0.reshape(NC, S0, N, 128), degp, x,
                   wa0, wx0, b0r, g0r, bn0r, ln_relu=True)

    p1 = _make_spmm(N, E, SH)(h0.reshape(-1, 128), src, dst, z128)
    h1 = _tc_layer(p1.reshape(NC, SH, N, 128), degp, h0,
                   wa1, wx1, b1r, g1r, bn1r, ln_relu=True)

    ya, yx = _tc_lin2(h1, wa2, wx2, b2r)
    p2 = _make_spmm(N, E, SO)(ya.reshape(-1, 128), src, dst, z128)
    out = _tc_final(p2.reshape(NC, SO, N, 128), degp, yx)
    return out


# speedup vs baseline: 1.4816x; 1.4816x over previous
import functools

import jax
import jax.numpy as jnp
from jax import lax
from jax.experimental import pallas as pl
from jax.experimental.pallas import tpu as pltpu
from jax.experimental.pallas import tpu_sc as plsc

EPS = 1e-5

NC = 2
NS = 16
NW = NC * NS

KC = 144
WT = 10


def _make_spmm(N, E, S, with_deg=False, interpret=False):
    EPT = E // NW
    FULL = EPT // KC
    TAILB = FULL * KC
    TAILN = EPT - TAILB
    CH = FULL + 1
    assert CH % 2 == 1 and TAILN % 16 == 8 and 0 < TAILN < KC
    RPT = N // WT
    mesh = plsc.VectorSubcoreMesh(core_axis_name="c", subcore_axis_name="s")

    parts_t = jax.ShapeDtypeStruct((NC * S * N, 128), jnp.float32)
    out_t = [parts_t, jax.ShapeDtypeStruct((NC * N, 128), jnp.float32)] \
        if with_deg else parts_t

    @functools.partial(
        pl.kernel,
        out_type=out_t,
        mesh=mesh,
        interpret=interpret,
        scratch_types=[
            pltpu.VMEM((EPT + 16,), jnp.int32),
            pltpu.VMEM((EPT + 16,), jnp.int32),
            pltpu.VMEM((KC,), jnp.int32),
            pltpu.VMEM((KC,), jnp.int32),
            pltpu.VMEM((KC,), jnp.int32),
            pltpu.VMEM((KC,), jnp.int32),
            pltpu.VMEM((KC, 128), jnp.float32),
            pltpu.VMEM((KC, 128), jnp.float32),
            pltpu.VMEM_SHARED((N + 8, 128), jnp.float32),
            pltpu.SemaphoreType.DMA,
            pltpu.SemaphoreType.DMA,
            pltpu.SemaphoreType.DMA,
            pltpu.SemaphoreType.DMA,
        ],
    )
    def spmm(table, src, dst, zeros, ones, *args):
        if with_deg:
            (out, dout, src_all, dst_all, gidxA, gidxB, sidxA, sidxB,
             rowsA, rowsB, acc, semGA, semGB, semSA, semSB) = args
        else:
            (out, src_all, dst_all, gidxA, gidxB, sidxA, sidxB,
             rowsA, rowsB, acc, semGA, semGB, semSA, semSB) = args
        cid = lax.axis_index("c")
        sid = lax.axis_index("s")
        wid = cid * NS + sid
        ebase = wid * EPT
        pltpu.sync_copy(src.at[pl.ds(ebase, EPT)], src_all.at[pl.ds(0, EPT)])
        pltpu.sync_copy(dst.at[pl.ds(ebase, EPT)], dst_all.at[pl.ds(0, EPT)])

        def zero_acc():
            plsc.subcore_barrier()

            @pl.when(sid < WT)
            def _():
                pltpu.sync_copy(zeros, acc.at[pl.ds(sid * RPT, RPT)])

            plsc.subcore_barrier()

        def writeout(obase):
            plsc.subcore_barrier()

            @pl.when(sid < WT)
            def _():
                pltpu.sync_copy(acc.at[pl.ds(sid * RPT, RPT)],
                                out.at[pl.ds(obase + sid * RPT, RPT)])

        def build_s(g, sidx):
            for i in range(KC // 16):
                o = g * KC + i * 16
                sidx[pl.ds(i * 16, 16)] = dst_all[pl.ds(o, 16)]

        def build_s_tail(sidx):
            lane = lax.iota(jnp.int32, 16)
            nf = TAILN // 16
            for i in range(nf):
                sidx[pl.ds(i * 16, 16)] = dst_all[pl.ds(TAILB + i * 16, 16)]
            m = lane < (TAILN - nf * 16)
            dv = dst_all[pl.ds(TAILB + nf * 16, 16)]
            sidx[pl.ds(nf * 16, 16)] = jnp.where(m, dv, N)
            for i in range(nf + 1, KC // 16):
                sidx[pl.ds(i * 16, 16)] = jnp.full((16,), N, jnp.int32)

        def build_g(g, gidx, s):
            for i in range(KC // 16):
                o = g * KC + i * 16
                gidx[pl.ds(i * 16, 16)] = src_all[pl.ds(o, 16)] * S + s

        def build_g_tail(gidx, s):
            lane = lax.iota(jnp.int32, 16)
            nf = TAILN // 16
            for i in range(nf):
                gidx[pl.ds(i * 16, 16)] = \
                    src_all[pl.ds(TAILB + i * 16, 16)] * S + s
            m = lane < (TAILN - nf * 16)
            sv = src_all[pl.ds(TAILB + nf * 16, 16)]
            gidx[pl.ds(nf * 16, 16)] = jnp.where(m, sv * S + s, 0)
            for i in range(nf + 1, KC // 16):
                gidx[pl.ds(i * 16, 16)] = jnp.zeros((16,), jnp.int32)

        def startG(gidx, rows, sem):
            pltpu.async_copy(table.at[gidx], rows, sem)

        def waitG(gidx, rows, sem):
            pltpu.make_async_copy(table.at[gidx], rows, sem).wait()

        def startS(rows, sidx, sem):
            pltpu.async_copy(rows, acc.at[sidx], sem, add=True)

        def waitS(rows, sidx, sem):
            pltpu.make_async_copy(rows, acc.at[sidx], sem).wait()

        for s in range(S):
            zero_acc()

            def bld(g, gidx, sidx):
                build_g(g, gidx, s)
                build_s(g, sidx)

            bld(0, gidxA, sidxA)
            startG(gidxA, rowsA, semGA)
            bld(1, gidxB, sidxB)
            startG(gidxB, rowsB, semGB)
            waitG(gidxA, rowsA, semGA)
            startS(rowsA, sidxA, semSA)

            def pair(i, _):
                g = 2 * i
                waitS(rowsA, sidxA, semSA)
                bld(g, gidxA, sidxA)
                startG(gidxA, rowsA, semGA)
                waitG(gidxB, rowsB, semGB)
                startS(rowsB, sidxB, semSB)
                waitG(gidxA, rowsA, semGA)
                startS(rowsA, sidxA, semSA)
                waitS(rowsB, sidxB, semSB)
                bld(g + 1, gidxB, sidxB)
                startG(gidxB, rowsB, semGB)
                return 0

            lax.fori_loop(1, (CH - 1) // 2, pair, 0)
            waitS(rowsA, sidxA, semSA)
            build_g_tail(gidxA, s)
            build_s_tail(sidxA)
            startG(gidxA, rowsA, semGA)
            waitG(gidxB, rowsB, semGB)
            startS(rowsB, sidxB, semSB)
            waitG(gidxA, rowsA, semGA)
            waitS(rowsB, sidxB, semSB)
            startS(rowsA, sidxA, semSA)
            waitS(rowsA, sidxA, semSA)

            writeout((cid * S + s) * N)

        if with_deg:
            def dwriteout(obase):
                plsc.subcore_barrier()

                @pl.when(sid < WT)
                def _():
                    pltpu.sync_copy(acc.at[pl.ds(sid * RPT, RPT)],
                                    dout.at[pl.ds(obase + sid * RPT, RPT)])

            zero_acc()
            pltpu.sync_copy(ones, rowsA)

            build_s(0, sidxA)
            startS(rowsA, sidxA, semSA)

            def dpair(i, _):
                g = 2 * i
                build_s(g - 1, sidxB)
                startS(rowsA, sidxB, semSB)
                waitS(rowsA, sidxA, semSA)
                build_s(g, sidxA)
                startS(rowsA, sidxA, semSA)
                waitS(rowsA, sidxB, semSB)
                return 0

            lax.fori_loop(1, (CH - 1) // 2, dpair, 0)
            build_s(CH - 2, sidxB)
            startS(rowsA, sidxB, semSB)
            waitS(rowsA, sidxA, semSA)
            build_s_tail(sidxA)
            startS(rowsA, sidxA, semSA)
            waitS(rowsA, sidxB, semSB)
            waitS(rowsA, sidxA, semSA)

            dwriteout(cid * N)

    return spmm


def _inv_deg(dp):
    deg = dp[0, :, 0:1] + dp[1, :, 0:1]
    return jnp.where(deg > 0.0, 1.0 / jnp.maximum(deg, 1.0), 0.0)


def _tc_layer(parts, degp, xin, wa, wx, b, g, bn, stage2=None, *, R=400,
              interpret=False):
    N, C = xin.shape
    S = C // 128
    H = wa.shape[1]

    def body(p_ref, d_ref, x_ref, wa_ref, wx_ref, b_ref, g_ref, bn_ref,
             *refs):
        p = p_ref[...]
        ps = p[0] + p[1]
        inv = _inv_deg(d_ref[...])
        agg = jnp.concatenate([ps[s] for s in range(S)], axis=-1) * inv
        h = (jnp.dot(agg, wa_ref[...], preferred_element_type=jnp.float32)
             + jnp.dot(x_ref[...], wx_ref[...],
                       preferred_element_type=jnp.float32)
             + b_ref[...])
        mu = jnp.mean(h, axis=-1, keepdims=True)
        var = jnp.mean((h - mu) ** 2, axis=-1, keepdims=True)
        h = (h - mu) * lax.rsqrt(var + EPS) * g_ref[...] + bn_ref[...]
        h = jnp.maximum(h, 0.0)
        if stage2 is None:
            refs[0][...] = h
        else:
            wa2_ref, wx2_ref, b2_ref, ya_ref, yx_ref = refs
            ya_ref[...] = jnp.dot(h, wa2_ref[...],
                                  preferred_element_type=jnp.float32)
            yx_ref[...] = jnp.dot(h, wx2_ref[...],
                                  preferred_element_type=jnp.float32) \
                + b2_ref[...]

    full = lambda i: (0, 0)
    in_specs = [
        pl.BlockSpec((NC, S, R, 128), lambda i: (0, 0, i, 0)),
        pl.BlockSpec((NC, R, 128), lambda i: (0, i, 0)),
        pl.BlockSpec((R, C), lambda i: (i, 0)),
        pl.BlockSpec((C, H), full),
        pl.BlockSpec((C, H), full),
        pl.BlockSpec((1, H), full),
        pl.BlockSpec((1, H), full),
        pl.BlockSpec((1, H), full),
    ]
    args = [parts, degp, xin, wa, wx, b, g, bn]
    if stage2 is None:
        out_specs = pl.BlockSpec((R, H), lambda i: (i, 0))
        out_shape = jax.ShapeDtypeStruct((N, H), jnp.float32)
    else:
        wa2, wx2, b2 = stage2
        H2 = wa2.shape[1]
        in_specs += [pl.BlockSpec((H, H2), full), pl.BlockSpec((H, H2), full),
                     pl.BlockSpec((1, H2), full)]
        args += [wa2, wx2, b2]
        out_specs = [pl.BlockSpec((R, H2), lambda i: (i, 0)),
                     pl.BlockSpec((R, H2), lambda i: (i, 0))]
        out_shape = [jax.ShapeDtypeStruct((N, H2), jnp.float32),
                     jax.ShapeDtypeStruct((N, H2), jnp.float32)]

    return pl.pallas_call(
        body,
        grid=(N // R,),
        in_specs=in_specs,
        out_specs=out_specs,
        out_shape=out_shape,
        interpret=interpret,
    )(*args)


def _tc_final(parts, degp, yx, *, R=400, interpret=False):
    N, H = yx.shape
    S = H // 128

    def body(p_ref, d_ref, y_ref, o_ref):
        p = p_ref[...]
        ps = p[0] + p[1]
        inv = _inv_deg(d_ref[...])
        agg = jnp.concatenate([ps[s] for s in range(S)], axis=-1) * inv
        o_ref[...] = agg + y_ref[...]

    return pl.pallas_call(
        body,
        grid=(N // R,),
        in_specs=[
            pl.BlockSpec((NC, S, R, 128), lambda i: (0, 0, i, 0)),
            pl.BlockSpec((NC, R, 128), lambda i: (0, i, 0)),
            pl.BlockSpec((R, H), lambda i: (i, 0)),
        ],
        out_specs=pl.BlockSpec((R, H), lambda i: (i, 0)),
        out_shape=jax.ShapeDtypeStruct((N, H), jnp.float32),
        interpret=interpret,
    )(parts, degp, yx)


def kernel(x, edge_index, W0, b0, W1, b1, W2, b2, g0, bn0, g1, bn1):
    N, C0 = x.shape
    E = edge_index.shape[1]
    H = W0.shape[0]
    OUT = W2.shape[0]
    src = edge_index[0].astype(jnp.int32)
    dst = edge_index[1].astype(jnp.int32)

    Wt0, Wt1, Wt2 = W0.T, W1.T, W2.T
    wa0, wx0 = Wt0[:C0], Wt0[C0:]
    wa1, wx1 = Wt1[:H], Wt1[H:]
    wa2, wx2 = Wt2[:H], Wt2[H:]
    b0r, g0r, bn0r = b0.reshape(1, -1), g0.reshape(1, -1), bn0.reshape(1, -1)
    b1r, g1r, bn1r = b1.reshape(1, -1), g1.reshape(1, -1), bn1.reshape(1, -1)
    b2r = b2.reshape(1, -1)

    zrows = jnp.zeros((N // WT, 128), jnp.float32)
    orows = jnp.ones((KC, 128), jnp.float32)

    S0 = C0 // 128
    SH = H // 128
    SO = OUT // 128

    p0, degp = _make_spmm(N, E, S0, with_deg=True)(
        x.reshape(-1, 128), src, dst, zrows, orows)
    degp = degp.reshape(NC, N, 128)
    h0 = _tc_layer(p0.reshape(NC, S0, N, 128), degp, x,
                   wa0, wx0, b0r, g0r, bn0r)

    p1 = _make_spmm(N, E, SH)(h0.reshape(-1, 128), src, dst, zrows, orows)
    ya, yx = _tc_layer(p1.reshape(NC, SH, N, 128), degp, h0,
                       wa1, wx1, b1r, g1r, bn1r, stage2=(wa2, wx2, b2r))

    p2 = _make_spmm(N, E, SO)(ya.reshape(-1, 128), src, dst, zrows, orows)
    out = _tc_final(p2.reshape(NC, SO, N, 128), degp, yx)
    return out


# speedup vs baseline: 1.5429x; 1.0414x over previous
import functools

import jax
import jax.numpy as jnp
from jax import lax
from jax.experimental import pallas as pl
from jax.experimental.pallas import tpu as pltpu
from jax.experimental.pallas import tpu_sc as plsc

EPS = 1e-5

NC = 2
NS = 16
NW = NC * NS

KC = 144
WT = 10


def _make_spmm(N, E, S, with_deg=False, interpret=False):
    EPT = E // NW
    FULL = EPT // KC
    TAILB = FULL * KC
    TAILN = EPT - TAILB
    CH = FULL + 1
    PAD = CH * KC - EPT
    assert CH % 2 == 1 and TAILN % 16 == 8
    RPT = N // WT
    mesh = plsc.VectorSubcoreMesh(core_axis_name="c", subcore_axis_name="s")

    parts_t = jax.ShapeDtypeStruct((NC * S * N, 128), jnp.float32)
    out_t = [parts_t, jax.ShapeDtypeStruct((NC * N, 128), jnp.float32)] \
        if with_deg else parts_t

    @functools.partial(
        pl.kernel,
        out_type=out_t,
        mesh=mesh,
        interpret=interpret,
        scratch_types=[
            pltpu.VMEM((CH * KC,), jnp.int32),
            pltpu.VMEM((EPT + 16,), jnp.int32),
            pltpu.VMEM((KC,), jnp.int32),
            pltpu.VMEM((KC,), jnp.int32),
            pltpu.VMEM((KC, 128), jnp.float32),
            pltpu.VMEM((KC, 128), jnp.float32),
            pltpu.VMEM_SHARED((N + 8, 128), jnp.float32),
            pltpu.SemaphoreType.DMA,
            pltpu.SemaphoreType.DMA,
            pltpu.SemaphoreType.DMA,
            pltpu.SemaphoreType.DMA,
        ],
    )
    def spmm(*refs):
        tabs = refs[:S]
        src, dst, zeros, ones = refs[S:S + 4]
        if with_deg:
            (out, dout, src_all, dst_all, sidxA, sidxB, rowsA, rowsB, acc,
             semGA, semGB, semSA, semSB) = refs[S + 4:]
        else:
            (out, src_all, dst_all, sidxA, sidxB, rowsA, rowsB, acc,
             semGA, semGB, semSA, semSB) = refs[S + 4:]
        cid = lax.axis_index("c")
        sid = lax.axis_index("s")
        wid = cid * NS + sid
        ebase = wid * EPT
        NB = KC * 128 * 4

        pltpu.sync_copy(src.at[pl.ds(ebase, EPT)], src_all.at[pl.ds(0, EPT)])
        pltpu.sync_copy(dst.at[pl.ds(ebase, EPT)], dst_all.at[pl.ds(0, EPT)])
        lane = lax.iota(jnp.int32, 16)
        nf = (TAILN // 16) * 16
        sv = src_all[pl.ds(TAILB + nf, 16)]
        src_all[pl.ds(TAILB + nf, 16)] = jnp.where(lane < TAILN - nf, sv, 0)
        for o in range(nf + 16, KC, 16):
            src_all[pl.ds(TAILB + o, 16)] = jnp.zeros((16,), jnp.int32)

        def build_s(g, sidx):
            for i in range(KC // 16):
                sidx[pl.ds(i * 16, 16)] = dst_all[pl.ds(g * KC + i * 16, 16)]

        def build_s_tail(sidx):
            for i in range(TAILN // 16):
                sidx[pl.ds(i * 16, 16)] = \
                    dst_all[pl.ds(TAILB + i * 16, 16)]
            dv = dst_all[pl.ds(TAILB + nf, 16)]
            sidx[pl.ds(nf, 16)] = jnp.where(lane < TAILN - nf, dv, N)
            for o in range(nf + 16, KC, 16):
                sidx[pl.ds(o, 16)] = jnp.full((16,), N, jnp.int32)

        def zero_acc():
            plsc.subcore_barrier()

            @pl.when(sid < WT)
            def _():
                pltpu.sync_copy(zeros, acc.at[pl.ds(sid * RPT, RPT)])

            plsc.subcore_barrier()

        def writeout(dest, obase):
            plsc.subcore_barrier()

            @pl.when(sid < WT)
            def _():
                pltpu.sync_copy(acc.at[pl.ds(sid * RPT, RPT)],
                                dest.at[pl.ds(obase + sid * RPT, RPT)])

        def startG(tab, c, rows, sem):
            pltpu.async_copy(tab.at[src_all.at[pl.ds(c * KC, KC)]],
                             rows, sem)

        def startS(rows, sidx, sem):
            pltpu.async_copy(rows, acc.at[sidx], sem, add=True)

        def wait(sem):
            pltpu.make_async_copy(tabs[0].at[pl.ds(0, KC)], rowsA,
                                  sem).wait()

        for s in range(S):
            tab = tabs[s]
            zero_acc()
            build_s(0, sidxA)
            startG(tab, 0, rowsA, semGA)
            build_s(1, sidxB)
            startG(tab, 1, rowsB, semGB)
            wait(semGA)
            startS(rowsA, sidxA, semSA)

            def pair(i, _):
                g = 2 * i
                wait(semSA)
                build_s(g, sidxA)
                startG(tab, g, rowsA, semGA)
                wait(semGB)
                startS(rowsB, sidxB, semSB)
                wait(semGA)
                startS(rowsA, sidxA, semSA)
                wait(semSB)
                build_s(g + 1, sidxB)
                startG(tab, g + 1, rowsB, semGB)
                return 0

            lax.fori_loop(1, (CH - 1) // 2, pair, 0)
            wait(semSA)
            build_s_tail(sidxA)
            startG(tab, CH - 1, rowsA, semGA)
            wait(semGB)
            startS(rowsB, sidxB, semSB)
            wait(semGA)
            wait(semSB)
            startS(rowsA, sidxA, semSA)
            wait(semSA)

            writeout(out, (cid * S + s) * N)

        if with_deg:
            zero_acc()
            pltpu.sync_copy(ones, rowsA)
            build_s(0, sidxA)
            startS(rowsA, sidxA, semSA)

            def dpair(i, _):
                g = 2 * i
                build_s(g - 1, sidxB)
                startS(rowsA, sidxB, semSB)
                wait(semSA)
                build_s(g, sidxA)
                startS(rowsA, sidxA, semSA)
                wait(semSB)
                return 0

            lax.fori_loop(1, (CH - 1) // 2, dpair, 0)
            build_s(CH - 2, sidxB)
            startS(rowsA, sidxB, semSB)
            wait(semSA)
            build_s_tail(sidxA)
            startS(rowsA, sidxA, semSA)
            wait(semSB)
            wait(semSA)

            writeout(dout, cid * N)

    return spmm


def _inv_deg(dp):
    deg = dp[0, :, 0:1] + dp[1, :, 0:1]
    return jnp.where(deg > 0.0, 1.0 / jnp.maximum(deg, 1.0), 0.0)


def _tc_layer(parts, degp, xins, wa, wx, b, g, bn, stage2=None, *, R=400,
              interpret=False):
    SX = len(xins)
    N = xins[0].shape[0]
    C = SX * 128
    S = parts.shape[1]
    H = wa.shape[1]

    def body(*refs):
        p_ref, d_ref = refs[0], refs[1]
        x_refs = refs[2:2 + SX]
        wa_ref, wx_ref, b_ref, g_ref, bn_ref = refs[2 + SX:7 + SX]
        rest = refs[7 + SX:]
        p = p_ref[...]
        ps = p[0] + p[1]
        inv = _inv_deg(d_ref[...])
        agg = jnp.concatenate([ps[s] for s in range(S)], axis=-1) * inv
        xcat = jnp.concatenate([x[...] for x in x_refs], axis=-1)
        h = (jnp.dot(agg, wa_ref[...], preferred_element_type=jnp.float32)
             + jnp.dot(xcat, wx_ref[...],
                       preferred_element_type=jnp.float32)
             + b_ref[...])
        mu = jnp.mean(h, axis=-1, keepdims=True)
        var = jnp.mean((h - mu) ** 2, axis=-1, keepdims=True)
        h = (h - mu) * lax.rsqrt(var + EPS) * g_ref[...] + bn_ref[...]
        h = jnp.maximum(h, 0.0)
        if stage2 is None:
            for t in range(H // 128):
                rest[t][...] = h[:, t * 128:(t + 1) * 128]
        else:
            wa2_ref, wx2_ref, b2_ref = rest[:3]
            ya = jnp.dot(h, wa2_ref[...], preferred_element_type=jnp.float32)
            H2 = wa2_ref.shape[1]
            for t in range(H2 // 128):
                rest[3 + t][...] = ya[:, t * 128:(t + 1) * 128]
            rest[3 + H2 // 128][...] = \
                jnp.dot(h, wx2_ref[...],
                        preferred_element_type=jnp.float32) + b2_ref[...]

    full = lambda i: (0, 0)
    row = lambda i: (i, 0)
    slab_spec = pl.BlockSpec((R, 128), row)
    in_specs = [
        pl.BlockSpec((NC, S, R, 128), lambda i: (0, 0, i, 0)),
        pl.BlockSpec((NC, R, 128), lambda i: (0, i, 0)),
    ] + [slab_spec] * SX + [
        pl.BlockSpec((C, H), full),
        pl.BlockSpec((C, H), full),
        pl.BlockSpec((1, H), full),
        pl.BlockSpec((1, H), full),
        pl.BlockSpec((1, H), full),
    ]
    args = [parts, degp] + list(xins) + [wa, wx, b, g, bn]
    slab_t = jax.ShapeDtypeStruct((N, 128), jnp.float32)
    if stage2 is None:
        out_specs = [slab_spec] * (H // 128)
        out_shape = [slab_t] * (H // 128)
    else:
        wa2, wx2, b2 = stage2
        H2 = wa2.shape[1]
        in_specs += [pl.BlockSpec((H, H2), full), pl.BlockSpec((H, H2), full),
                     pl.BlockSpec((1, H2), full)]
        args += [wa2, wx2, b2]
        out_specs = [slab_spec] * (H2 // 128) + [pl.BlockSpec((R, H2), row)]
        out_shape = [slab_t] * (H2 // 128) + \
            [jax.ShapeDtypeStruct((N, H2), jnp.float32)]

    res = pl.pallas_call(
        body,
        grid=(N // R,),
        in_specs=in_specs,
        out_specs=out_specs,
        out_shape=out_shape,
        interpret=interpret,
    )(*args)
    if stage2 is None:
        return res
    return res[:-1], res[-1]


def _tc_final(parts, degp, yx, *, R=400, interpret=False):
    N, H = yx.shape
    S = H // 128

    def body(p_ref, d_ref, y_ref, o_ref):
        p = p_ref[...]
        ps = p[0] + p[1]
        inv = _inv_deg(d_ref[...])
        agg = jnp.concatenate([ps[s] for s in range(S)], axis=-1) * inv
        o_ref[...] = agg + y_ref[...]

    return pl.pallas_call(
        body,
        grid=(N // R,),
        in_specs=[
            pl.BlockSpec((NC, S, R, 128), lambda i: (0, 0, i, 0)),
            pl.BlockSpec((NC, R, 128), lambda i: (0, i, 0)),
            pl.BlockSpec((R, H), lambda i: (i, 0)),
        ],
        out_specs=pl.BlockSpec((R, H), lambda i: (i, 0)),
        out_shape=jax.ShapeDtypeStruct((N, H), jnp.float32),
        interpret=interpret,
    )(parts, degp, yx)


def kernel(x, edge_index, W0, b0, W1, b1, W2, b2, g0, bn0, g1, bn1):
    N, C0 = x.shape
    E = edge_index.shape[1]
    H = W0.shape[0]
    src = edge_index[0].astype(jnp.int32)
    dst = edge_index[1].astype(jnp.int32)

    Wt0, Wt1, Wt2 = W0.T, W1.T, W2.T
    wa0, wx0 = Wt0[:C0], Wt0[C0:]
    wa1, wx1 = Wt1[:H], Wt1[H:]
    wa2, wx2 = Wt2[:H], Wt2[H:]
    b0r, g0r, bn0r = b0.reshape(1, -1), g0.reshape(1, -1), bn0.reshape(1, -1)
    b1r, g1r, bn1r = b1.reshape(1, -1), g1.reshape(1, -1), bn1.reshape(1, -1)
    b2r = b2.reshape(1, -1)

    zrows = jnp.zeros((N // WT, 128), jnp.float32)
    orows = jnp.ones((KC, 128), jnp.float32)

    S0 = C0 // 128
    x_slabs = [x[:, 128 * s:128 * (s + 1)] for s in range(S0)]

    p0, degp = _make_spmm(N, E, S0, with_deg=True)(
        *x_slabs, src, dst, zrows, orows)
    degp = degp.reshape(NC, N, 128)
    h0_slabs = _tc_layer(p0.reshape(NC, S0, N, 128), degp, x_slabs,
                         wa0, wx0, b0r, g0r, bn0r)

    SH = H // 128
    p1 = _make_spmm(N, E, SH)(*h0_slabs, src, dst, zrows, orows)
    ya_slabs, yx = _tc_layer(p1.reshape(NC, SH, N, 128), degp, h0_slabs,
                             wa1, wx1, b1r, g1r, bn1r,
                             stage2=(wa2, wx2, b2r))

    SO = len(ya_slabs)
    p2 = _make_spmm(N, E, SO)(*ya_slabs, src, dst, zrows, orows)
    out = _tc_final(p2.reshape(NC, SO, N, 128), degp, yx)
    return out
